# baseline (speedup 1.0000x reference)
import functools

import jax
import jax.numpy as jnp
from jax import lax
from jax.experimental import pallas as pl
from jax.experimental.pallas import tpu as pltpu
from jax.experimental.pallas import tpu_sc as plsc

N = 10000
D = 128
NPAD = 10240
E = 320000
NW = 32
CHUNK = 64
CH = 160
NBUF = 4
PH = 4
CPP = CH // PH
EPAD = NW * CH * CHUNK
ROWS_PER_TILE = NPAD // 16
BM = 512
GRID = NPAD // BM

_mesh = plsc.VectorSubcoreMesh(core_axis_name="c", subcore_axis_name="s")


DEGW = 8

@functools.partial(
    pl.kernel,
    out_type=jax.ShapeDtypeStruct((2, NPAD, D), jnp.float32),
    mesh=_mesh,
    scratch_types=[
        pltpu.VMEM((CH, CHUNK), jnp.int32),
        pltpu.VMEM((CHUNK, D), jnp.float32),
        pltpu.VMEM_SHARED((NPAD, D), jnp.float32),
    ]
    + [pltpu.SemaphoreType.DMA] * DEGW,
)
def _sc_deg(dst_hbm, out_hbm, idx_v, ones_v, acc, *sems):
    c = lax.axis_index("c")
    s = lax.axis_index("s")
    wid = c * 16 + s
    ones16 = jnp.full((16,), 1.0, jnp.float32)
    zero16 = jnp.zeros((16,), jnp.float32)

    @pl.loop(0, CHUNK)
    def _(i):
        for j in range(D // 16):
            ones_v[i, pl.ds(j * 16, 16)] = zero16

    for k in range(ROWS_PER_TILE // CHUNK):
        pltpu.sync_copy(ones_v, acc.at[pl.ds(s * ROWS_PER_TILE + k * CHUNK, CHUNK)])
    pltpu.sync_copy(dst_hbm.at[wid], idx_v)

    @pl.loop(0, CHUNK)
    def _(i):
        for j in range(D // 16):
            ones_v[i, pl.ds(j * 16, 16)] = ones16

    plsc.subcore_barrier()

    for b in range(DEGW):
        pltpu.async_copy(ones_v, acc.at[idx_v.at[b]], sems[b], add=True)

    @pl.loop(0, CH - DEGW, step=DEGW)
    def _(g):
        for b in range(DEGW):
            pltpu.make_async_copy(ones_v, acc.at[idx_v.at[g + b]], sems[b]).wait()
            pltpu.async_copy(
                ones_v, acc.at[idx_v.at[g + DEGW + b]], sems[b], add=True
            )

    for b in range(DEGW):
        pltpu.make_async_copy(
            ones_v, acc.at[idx_v.at[CH - DEGW + b]], sems[b]
        ).wait()

    plsc.subcore_barrier()
    pltpu.sync_copy(
        acc.at[pl.ds(s * ROWS_PER_TILE, ROWS_PER_TILE)],
        out_hbm.at[c, pl.ds(s * ROWS_PER_TILE, ROWS_PER_TILE)],
    )


@functools.partial(
    pl.kernel,
    out_type=jax.ShapeDtypeStruct((2, NPAD, D), jnp.float32),
    mesh=_mesh,
    scratch_types=[
        pltpu.VMEM((CPP, CHUNK), jnp.int32),
        pltpu.VMEM((CPP, CHUNK), jnp.int32),
        pltpu.VMEM((NBUF, CHUNK, D), jnp.float32),
        pltpu.VMEM_SHARED((NPAD, D), jnp.float32),
    ]
    + [pltpu.SemaphoreType.DMA] * (2 * NBUF),
)
def _sc_seg(sup_hbm, src_hbm, dst_hbm, out_hbm, srcv, dstv, rows, acc, *sems):
    c = lax.axis_index("c")
    s = lax.axis_index("s")
    wid = c * 16 + s
    zero16 = jnp.zeros((16,), jnp.float32)

    @pl.loop(0, CHUNK)
    def _(i):
        for j in range(D // 16):
            rows[0, i, pl.ds(j * 16, 16)] = zero16

    for k in range(ROWS_PER_TILE // CHUNK):
        pltpu.sync_copy(
            rows.at[0], acc.at[pl.ds(s * ROWS_PER_TILE + k * CHUNK, CHUNK)]
        )
    plsc.subcore_barrier()

    gsem = sems[:NBUF]
    ssem = sems[NBUF:]
    for p in range(PH):
        pltpu.sync_copy(src_hbm.at[wid, pl.ds(p * CPP, CPP)], srcv)
        pltpu.sync_copy(dst_hbm.at[wid, pl.ds(p * CPP, CPP)], dstv)

        for b in range(NBUF):
            pltpu.async_copy(sup_hbm.at[srcv.at[b]], rows.at[b], gsem[b])

        @pl.loop(0, CPP - NBUF, step=NBUF)
        def _(g):
            for b in range(NBUF):
                cur = g + b
                pltpu.make_async_copy(
                    sup_hbm.at[srcv.at[cur]], rows.at[b], gsem[b]
                ).wait()
                pltpu.async_copy(rows.at[b], acc.at[dstv.at[cur]], ssem[b], add=True)
            for b in range(NBUF):
                cur = g + b
                pltpu.make_async_copy(rows.at[b], acc.at[dstv.at[cur]], ssem[b]).wait()
                pltpu.async_copy(sup_hbm.at[srcv.at[cur + NBUF]], rows.at[b], gsem[b])

        for b in range(NBUF):
            cur = CPP - NBUF + b
            pltpu.make_async_copy(sup_hbm.at[srcv.at[cur]], rows.at[b], gsem[b]).wait()
            pltpu.async_copy(rows.at[b], acc.at[dstv.at[cur]], ssem[b], add=True)
        for b in range(NBUF):
            cur = CPP - NBUF + b
            pltpu.make_async_copy(rows.at[b], acc.at[dstv.at[cur]], ssem[b]).wait()

    plsc.subcore_barrier()
    pltpu.sync_copy(
        acc.at[pl.ds(s * ROWS_PER_TILE, ROWS_PER_TILE)],
        out_hbm.at[c, pl.ds(s * ROWS_PER_TILE, ROWS_PER_TILE)],
    )


def _a_vec(d0, d1):
    deg = d0[:, :1] + d1[:, :1]
    return lax.rsqrt(jnp.maximum(deg, 1.0))


def _row_mask(t):
    rid = pl.program_id(0) * BM + lax.broadcasted_iota(jnp.int32, (BM, D), 0)
    return jnp.where(rid < N, t, 0.0)


def _tc_mm1(x_ref, w_ref, d0_ref, d1_ref, o_ref):
    a = _a_vec(d0_ref[...], d1_ref[...])
    t = jnp.dot(x_ref[...], w_ref[...], preferred_element_type=jnp.float32)
    o_ref[...] = _row_mask(t * a)


def _tc_mm2(g0_ref, g1_ref, d0_ref, d1_ref, b_ref, w_ref, o_ref):
    a = _a_vec(d0_ref[...], d1_ref[...])
    h = jnp.maximum((g0_ref[...] + g1_ref[...]) * a + b_ref[...], 0.0)
    t = jnp.dot(h, w_ref[...], preferred_element_type=jnp.float32)
    o_ref[...] = _row_mask(t * a)


def _tc_out(g0_ref, g1_ref, d0_ref, d1_ref, b_ref, o_ref):
    a = _a_vec(d0_ref[...], d1_ref[...])
    o_ref[...] = jnp.maximum((g0_ref[...] + g1_ref[...]) * a + b_ref[...], 0.0)


_bm_spec = pl.BlockSpec((BM, D), lambda i: (i, 0))
_d_spec = pl.BlockSpec((BM, D), lambda i: (i, 0))
_w_spec = pl.BlockSpec((D, D), lambda i: (0, 0))
_b_spec = pl.BlockSpec((1, D), lambda i: (0, 0))
_o_sds = jax.ShapeDtypeStruct((NPAD, D), jnp.float32)

_mm1 = pl.pallas_call(
    _tc_mm1,
    grid=(GRID,),
    in_specs=[_bm_spec, _w_spec, _d_spec, _d_spec],
    out_specs=_bm_spec,
    out_shape=_o_sds,
)
_mm2 = pl.pallas_call(
    _tc_mm2,
    grid=(GRID,),
    in_specs=[_bm_spec, _bm_spec, _d_spec, _d_spec, _b_spec, _w_spec],
    out_specs=_bm_spec,
    out_shape=_o_sds,
)
_outk = pl.pallas_call(
    _tc_out,
    grid=(GRID,),
    in_specs=[_bm_spec, _bm_spec, _d_spec, _d_spec, _b_spec],
    out_specs=_bm_spec,
    out_shape=_o_sds,
)


def kernel(x, edge_index, W1, b1, W2, b2, W3, b3):
    src = edge_index[0].astype(jnp.int32)
    dst = edge_index[1].astype(jnp.int32)
    pad = N + (jnp.arange(EPAD - E, dtype=jnp.int32) % (NPAD - N))
    src_p = jnp.concatenate([src, pad]).reshape(NW, CH, CHUNK)
    dst_p = jnp.concatenate([dst, pad]).reshape(NW, CH, CHUNK)
    x_p = jnp.pad(x, ((0, NPAD - N), (0, 0)))
    b1r = b1.reshape(1, D)
    b2r = b2.reshape(1, D)
    b3r = b3.reshape(1, D)

    d = _sc_deg(dst_p)
    d0, d1 = d[0], d[1]

    t = _mm1(x_p, W1, d0, d1)
    g = _sc_seg(t, src_p, dst_p)
    t = _mm2(g[0], g[1], d0, d1, b1r, W2)
    g = _sc_seg(t, src_p, dst_p)
    t = _mm2(g[0], g[1], d0, d1, b2r, W3)
    g = _sc_seg(t, src_p, dst_p)
    out = _outk(g[0], g[1], d0, d1, b3r)
    return out[:N]


# speedup vs baseline: 1.1036x; 1.1036x over previous
import functools

import jax
import jax.numpy as jnp
from jax import lax
from jax.experimental import pallas as pl
from jax.experimental.pallas import tpu as pltpu
from jax.experimental.pallas import tpu_sc as plsc

N = 10000
D = 128
NPAD = 10240
E = 320000
NW = 32
CHUNK = 64
CH = 160
NBUF = 4
PH = 4
CPP = CH // PH
EPAD = NW * CH * CHUNK
ROWS_PER_TILE = NPAD // 16
BM = 512
GRID = NPAD // BM

_mesh = plsc.VectorSubcoreMesh(core_axis_name="c", subcore_axis_name="s")


DEGW = 8

@functools.partial(
    pl.kernel,
    out_type=jax.ShapeDtypeStruct((2, NPAD, D), jnp.float32),
    mesh=_mesh,
    scratch_types=[
        pltpu.VMEM((CH, CHUNK), jnp.int32),
        pltpu.VMEM((CHUNK, D), jnp.float32),
        pltpu.VMEM_SHARED((NPAD, D), jnp.float32),
    ]
    + [pltpu.SemaphoreType.DMA] * DEGW,
)
def _sc_deg(dst_hbm, out_hbm, idx_v, ones_v, acc, *sems):
    c = lax.axis_index("c")
    s = lax.axis_index("s")
    wid = c * 16 + s
    ones16 = jnp.full((16,), 1.0, jnp.float32)
    zero16 = jnp.zeros((16,), jnp.float32)

    @pl.loop(0, CHUNK)
    def _(i):
        for j in range(D // 16):
            ones_v[i, pl.ds(j * 16, 16)] = zero16

    for k in range(ROWS_PER_TILE // CHUNK):
        pltpu.sync_copy(ones_v, acc.at[pl.ds(s * ROWS_PER_TILE + k * CHUNK, CHUNK)])
    pltpu.sync_copy(dst_hbm.at[wid], idx_v)

    @pl.loop(0, CHUNK)
    def _(i):
        for j in range(D // 16):
            ones_v[i, pl.ds(j * 16, 16)] = ones16

    plsc.subcore_barrier()

    for b in range(DEGW):
        pltpu.async_copy(ones_v, acc.at[idx_v.at[b]], sems[b], add=True)

    @pl.loop(0, CH - DEGW, step=DEGW)
    def _(g):
        for b in range(DEGW):
            pltpu.make_async_copy(ones_v, acc.at[idx_v.at[g + b]], sems[b]).wait()
            pltpu.async_copy(
                ones_v, acc.at[idx_v.at[g + DEGW + b]], sems[b], add=True
            )

    for b in range(DEGW):
        pltpu.make_async_copy(
            ones_v, acc.at[idx_v.at[CH - DEGW + b]], sems[b]
        ).wait()

    plsc.subcore_barrier()
    pltpu.sync_copy(
        acc.at[pl.ds(s * ROWS_PER_TILE, ROWS_PER_TILE)],
        out_hbm.at[c, pl.ds(s * ROWS_PER_TILE, ROWS_PER_TILE)],
    )


@functools.partial(
    pl.kernel,
    out_type=jax.ShapeDtypeStruct((2, NPAD, D), jnp.float32),
    mesh=_mesh,
    scratch_types=[
        pltpu.VMEM((CPP, CHUNK), jnp.int32),
        pltpu.VMEM((CPP, CHUNK), jnp.int32),
        pltpu.VMEM((NBUF, CHUNK, D), jnp.float32),
        pltpu.VMEM_SHARED((NPAD, D), jnp.float32),
    ]
    + [pltpu.SemaphoreType.DMA] * NBUF,
)
def _sc_seg(sup_hbm, src_hbm, dst_hbm, out_hbm, srcv, dstv, rows, acc, *sems):
    c = lax.axis_index("c")
    s = lax.axis_index("s")
    wid = c * 16 + s
    zero16 = jnp.zeros((16,), jnp.float32)

    @pl.loop(0, CHUNK)
    def _(i):
        for j in range(D // 16):
            rows[0, i, pl.ds(j * 16, 16)] = zero16

    for k in range(ROWS_PER_TILE // CHUNK):
        pltpu.sync_copy(
            rows.at[0], acc.at[pl.ds(s * ROWS_PER_TILE + k * CHUNK, CHUNK)]
        )
    plsc.subcore_barrier()

    for p in range(PH):
        pltpu.sync_copy(src_hbm.at[wid, pl.ds(p * CPP, CPP)], srcv)
        pltpu.sync_copy(dst_hbm.at[wid, pl.ds(p * CPP, CPP)], dstv)

        for b in range(NBUF):
            pltpu.async_copy(sup_hbm.at[srcv.at[b]], rows.at[b], sems[b])

        @pl.loop(0, CPP - NBUF, step=NBUF)
        def _(g):
            for b in range(NBUF):
                cur = g + b
                pltpu.make_async_copy(
                    sup_hbm.at[srcv.at[cur]], rows.at[b], sems[b]
                ).wait()
                pltpu.sync_copy(rows.at[b], acc.at[dstv.at[cur]], add=True)
                pltpu.async_copy(sup_hbm.at[srcv.at[cur + NBUF]], rows.at[b], sems[b])

        for b in range(NBUF):
            cur = CPP - NBUF + b
            pltpu.make_async_copy(sup_hbm.at[srcv.at[cur]], rows.at[b], sems[b]).wait()
            pltpu.sync_copy(rows.at[b], acc.at[dstv.at[cur]], add=True)

    plsc.subcore_barrier()
    pltpu.sync_copy(
        acc.at[pl.ds(s * ROWS_PER_TILE, ROWS_PER_TILE)],
        out_hbm.at[c, pl.ds(s * ROWS_PER_TILE, ROWS_PER_TILE)],
    )


def _a_vec(d0, d1):
    deg = d0[:, :1] + d1[:, :1]
    return lax.rsqrt(jnp.maximum(deg, 1.0))


def _row_mask(t):
    rid = pl.program_id(0) * BM + lax.broadcasted_iota(jnp.int32, (BM, D), 0)
    return jnp.where(rid < N, t, 0.0)


def _tc_mm1(x_ref, w_ref, d0_ref, d1_ref, o_ref):
    a = _a_vec(d0_ref[...], d1_ref[...])
    t = jnp.dot(x_ref[...], w_ref[...], preferred_element_type=jnp.float32)
    o_ref[...] = _row_mask(t * a)


def _tc_mm2(g0_ref, g1_ref, d0_ref, d1_ref, b_ref, w_ref, o_ref):
    a = _a_vec(d0_ref[...], d1_ref[...])
    h = jnp.maximum((g0_ref[...] + g1_ref[...]) * a + b_ref[...], 0.0)
    t = jnp.dot(h, w_ref[...], preferred_element_type=jnp.float32)
    o_ref[...] = _row_mask(t * a)


def _tc_out(g0_ref, g1_ref, d0_ref, d1_ref, b_ref, o_ref):
    a = _a_vec(d0_ref[...], d1_ref[...])
    o_ref[...] = jnp.maximum((g0_ref[...] + g1_ref[...]) * a + b_ref[...], 0.0)


_bm_spec = pl.BlockSpec((BM, D), lambda i: (i, 0))
_d_spec = pl.BlockSpec((BM, D), lambda i: (i, 0))
_w_spec = pl.BlockSpec((D, D), lambda i: (0, 0))
_b_spec = pl.BlockSpec((1, D), lambda i: (0, 0))
_o_sds = jax.ShapeDtypeStruct((NPAD, D), jnp.float32)

_mm1 = pl.pallas_call(
    _tc_mm1,
    grid=(GRID,),
    in_specs=[_bm_spec, _w_spec, _d_spec, _d_spec],
    out_specs=_bm_spec,
    out_shape=_o_sds,
)
_mm2 = pl.pallas_call(
    _tc_mm2,
    grid=(GRID,),
    in_specs=[_bm_spec, _bm_spec, _d_spec, _d_spec, _b_spec, _w_spec],
    out_specs=_bm_spec,
    out_shape=_o_sds,
)
_outk = pl.pallas_call(
    _tc_out,
    grid=(GRID,),
    in_specs=[_bm_spec, _bm_spec, _d_spec, _d_spec, _b_spec],
    out_specs=_bm_spec,
    out_shape=_o_sds,
)


def kernel(x, edge_index, W1, b1, W2, b2, W3, b3):
    src = edge_index[0].astype(jnp.int32)
    dst = edge_index[1].astype(jnp.int32)
    pad = N + (jnp.arange(EPAD - E, dtype=jnp.int32) % (NPAD - N))
    src_p = jnp.concatenate([src, pad]).reshape(NW, CH, CHUNK)
    dst_p = jnp.concatenate([dst, pad]).reshape(NW, CH, CHUNK)
    x_p = jnp.pad(x, ((0, NPAD - N), (0, 0)))
    b1r = b1.reshape(1, D)
    b2r = b2.reshape(1, D)
    b3r = b3.reshape(1, D)

    d = _sc_deg(dst_p)
    d0, d1 = d[0], d[1]

    t = _mm1(x_p, W1, d0, d1)
    g = _sc_seg(t, src_p, dst_p)
    t = _mm2(g[0], g[1], d0, d1, b1r, W2)
    g = _sc_seg(t, src_p, dst_p)
    t = _mm2(g[0], g[1], d0, d1, b2r, W3)
    g = _sc_seg(t, src_p, dst_p)
    out = _outk(g[0], g[1], d0, d1, b3r)
    return out[:N]


# speedup vs baseline: 1.1041x; 1.0005x over previous
import functools

import jax
import jax.numpy as jnp
from jax import lax
from jax.experimental import pallas as pl
from jax.experimental.pallas import tpu as pltpu
from jax.experimental.pallas import tpu_sc as plsc

N = 10000
D = 128
NPAD = 10240
E = 320000
NW = 32
CHUNK = 64
CH = 160
NBUF = 4
PH = 4
CPP = CH // PH
EPAD = NW * CH * CHUNK
ROWS_PER_TILE = NPAD // 16
BM = 512
GRID = NPAD // BM

_mesh = plsc.VectorSubcoreMesh(core_axis_name="c", subcore_axis_name="s")


DEGW = 8

@functools.partial(
    pl.kernel,
    out_type=jax.ShapeDtypeStruct((2, NPAD, D), jnp.float32),
    mesh=_mesh,
    scratch_types=[
        pltpu.VMEM((CH, CHUNK), jnp.int32),
        pltpu.VMEM((CHUNK, D), jnp.float32),
        pltpu.VMEM_SHARED((NPAD, D), jnp.float32),
    ]
    + [pltpu.SemaphoreType.DMA] * DEGW,
)
def _sc_deg(dst_hbm, out_hbm, idx_v, ones_v, acc, *sems):
    c = lax.axis_index("c")
    s = lax.axis_index("s")
    wid = c * 16 + s
    ones16 = jnp.full((16,), 1.0, jnp.float32)
    zero16 = jnp.zeros((16,), jnp.float32)

    @pl.loop(0, CHUNK)
    def _(i):
        for j in range(D // 16):
            ones_v[i, pl.ds(j * 16, 16)] = zero16

    for k in range(ROWS_PER_TILE // CHUNK):
        pltpu.sync_copy(ones_v, acc.at[pl.ds(s * ROWS_PER_TILE + k * CHUNK, CHUNK)])
    pltpu.sync_copy(dst_hbm.at[wid], idx_v)

    @pl.loop(0, CHUNK)
    def _(i):
        for j in range(D // 16):
            ones_v[i, pl.ds(j * 16, 16)] = ones16

    plsc.subcore_barrier()

    for b in range(DEGW):
        pltpu.async_copy(ones_v, acc.at[idx_v.at[b]], sems[b], add=True)

    @pl.loop(0, CH - DEGW, step=DEGW)
    def _(g):
        for b in range(DEGW):
            pltpu.make_async_copy(ones_v, acc.at[idx_v.at[g + b]], sems[b]).wait()
            pltpu.async_copy(
                ones_v, acc.at[idx_v.at[g + DEGW + b]], sems[b], add=True
            )

    for b in range(DEGW):
        pltpu.make_async_copy(
            ones_v, acc.at[idx_v.at[CH - DEGW + b]], sems[b]
        ).wait()

    plsc.subcore_barrier()
    pltpu.sync_copy(
        acc.at[pl.ds(s * ROWS_PER_TILE, ROWS_PER_TILE)],
        out_hbm.at[c, pl.ds(s * ROWS_PER_TILE, ROWS_PER_TILE)],
    )


@functools.partial(
    pl.kernel,
    out_type=jax.ShapeDtypeStruct((2, NPAD, D), jnp.float32),
    mesh=_mesh,
    scratch_types=[
        pltpu.VMEM((CPP, CHUNK), jnp.int32),
        pltpu.VMEM((CPP, CHUNK), jnp.int32),
        pltpu.VMEM((NBUF, CHUNK, D), jnp.float32),
        pltpu.VMEM_SHARED((NPAD, D), jnp.float32),
    ]
    + [pltpu.SemaphoreType.DMA] * NBUF,
)
def _sc_seg(sup_hbm, src_hbm, dst_hbm, out_hbm, srcv, dstv, rows, acc, *sems):
    c = lax.axis_index("c")
    s = lax.axis_index("s")
    wid = c * 16 + s
    zero16 = jnp.zeros((16,), jnp.float32)

    @pl.loop(0, CHUNK)
    def _(i):
        for j in range(D // 16):
            rows[0, i, pl.ds(j * 16, 16)] = zero16

    for k in range(ROWS_PER_TILE // CHUNK):
        pltpu.sync_copy(
            rows.at[0], acc.at[pl.ds(s * ROWS_PER_TILE + k * CHUNK, CHUNK)]
        )
    plsc.subcore_barrier()

    for p in range(PH):
        pltpu.sync_copy(src_hbm.at[wid, pl.ds(p * CPP, CPP)], srcv)
        pltpu.sync_copy(dst_hbm.at[wid, pl.ds(p * CPP, CPP)], dstv)

        for b in range(NBUF):
            pltpu.async_copy(sup_hbm.at[srcv.at[b]], rows.at[b], sems[b])

        @pl.loop(0, CPP - NBUF, step=NBUF)
        def _(g):
            for b in range(NBUF):
                cur = g + b
                pltpu.make_async_copy(
                    sup_hbm.at[srcv.at[cur]], rows.at[b], sems[b]
                ).wait()
                pltpu.sync_copy(rows.at[b], acc.at[dstv.at[cur]], add=True)
                pltpu.async_copy(sup_hbm.at[srcv.at[cur + NBUF]], rows.at[b], sems[b])

        for b in range(NBUF):
            cur = CPP - NBUF + b
            pltpu.make_async_copy(sup_hbm.at[srcv.at[cur]], rows.at[b], sems[b]).wait()
            pltpu.sync_copy(rows.at[b], acc.at[dstv.at[cur]], add=True)

    plsc.subcore_barrier()
    pltpu.sync_copy(
        acc.at[pl.ds(s * ROWS_PER_TILE, ROWS_PER_TILE)],
        out_hbm.at[c, pl.ds(s * ROWS_PER_TILE, ROWS_PER_TILE)],
    )


def _a_vec(d0, d1):
    deg = d0 + d1
    return lax.rsqrt(jnp.maximum(deg, 1.0))


def _row_mask(t):
    rid = pl.program_id(0) * BM + lax.broadcasted_iota(jnp.int32, (BM, D), 0)
    return jnp.where(rid < N, t, 0.0)


def _tc_mm1(x_ref, w_ref, d0_ref, d1_ref, o_ref):
    a = _a_vec(d0_ref[...], d1_ref[...])
    t = jnp.dot(x_ref[...], w_ref[...], preferred_element_type=jnp.float32)
    o_ref[...] = _row_mask(t * a)


def _tc_mm2(g0_ref, g1_ref, d0_ref, d1_ref, b_ref, w_ref, o_ref):
    a = _a_vec(d0_ref[...], d1_ref[...])
    h = jnp.maximum((g0_ref[...] + g1_ref[...]) * a + b_ref[...], 0.0)
    t = jnp.dot(h, w_ref[...], preferred_element_type=jnp.float32)
    o_ref[...] = _row_mask(t * a)


def _tc_out(g0_ref, g1_ref, d0_ref, d1_ref, b_ref, o_ref):
    a = _a_vec(d0_ref[...], d1_ref[...])
    o_ref[...] = jnp.maximum((g0_ref[...] + g1_ref[...]) * a + b_ref[...], 0.0)


_bm_spec = pl.BlockSpec((BM, D), lambda i: (i, 0))
_d_spec = pl.BlockSpec((BM, 1), lambda i: (i, 0))
_w_spec = pl.BlockSpec((D, D), lambda i: (0, 0))
_b_spec = pl.BlockSpec((1, D), lambda i: (0, 0))
_o_sds = jax.ShapeDtypeStruct((NPAD, D), jnp.float32)

_mm1 = pl.pallas_call(
    _tc_mm1,
    grid=(GRID,),
    in_specs=[_bm_spec, _w_spec, _d_spec, _d_spec],
    out_specs=_bm_spec,
    out_shape=_o_sds,
)
_mm2 = pl.pallas_call(
    _tc_mm2,
    grid=(GRID,),
    in_specs=[_bm_spec, _bm_spec, _d_spec, _d_spec, _b_spec, _w_spec],
    out_specs=_bm_spec,
    out_shape=_o_sds,
)
_outk = pl.pallas_call(
    _tc_out,
    grid=(GRID,),
    in_specs=[_bm_spec, _bm_spec, _d_spec, _d_spec, _b_spec],
    out_specs=_bm_spec,
    out_shape=_o_sds,
)


def kernel(x, edge_index, W1, b1, W2, b2, W3, b3):
    src = edge_index[0].astype(jnp.int32)
    dst = edge_index[1].astype(jnp.int32)
    pad = N + (jnp.arange(EPAD - E, dtype=jnp.int32) % (NPAD - N))
    src_p = jnp.concatenate([src, pad]).reshape(NW, CH, CHUNK)
    dst_p = jnp.concatenate([dst, pad]).reshape(NW, CH, CHUNK)
    x_p = jnp.pad(x, ((0, NPAD - N), (0, 0)))
    b1r = b1.reshape(1, D)
    b2r = b2.reshape(1, D)
    b3r = b3.reshape(1, D)

    d = _sc_deg(dst_p)
    d0 = lax.slice(d[0], (0, 0), (NPAD, 1))
    d1 = lax.slice(d[1], (0, 0), (NPAD, 1))

    t = _mm1(x_p, W1, d0, d1)
    g = _sc_seg(t, src_p, dst_p)
    t = _mm2(g[0], g[1], d0, d1, b1r, W2)
    g = _sc_seg(t, src_p, dst_p)
    t = _mm2(g[0], g[1], d0, d1, b2r, W3)
    g = _sc_seg(t, src_p, dst_p)
    out = _outk(g[0], g[1], d0, d1, b3r)
    return out[:N]
